# baseline (speedup 1.0000x reference)
import functools

import jax
import jax.numpy as jnp
from jax import lax
from jax.experimental import pallas as pl
from jax.experimental.pallas import tpu as pltpu
from jax.experimental.pallas import tpu_sc as plsc

_B = 4096
_DIM = 4096
_QD = 1024
_H = 512
_NK = 1024
_TEMP = 0.07
_K = 4
_NCAND = 16
_BB = 256

_PREC = lax.Precision.DEFAULT


def _keys_body(rk_ref, ck_ref, wr_ref, wc_ref, k1_ref, k2_ref):
    pk1 = lax.dot_general(rk_ref[...], wr_ref[...], (((1,), (1,)), ((), ())),
                          precision=_PREC, preferred_element_type=jnp.float32)
    n1 = jnp.sqrt(jnp.sum(pk1 * pk1, axis=1, keepdims=True))
    k1_ref[...] = pk1 / jnp.maximum(n1, 1e-12)
    pk2 = lax.dot_general(ck_ref[...], wc_ref[...], (((1,), (1,)), ((), ())),
                          precision=_PREC, preferred_element_type=jnp.float32)
    n2 = jnp.sqrt(jnp.sum(pk2 * pk2, axis=1, keepdims=True))
    k2_ref[...] = pk2 / jnp.maximum(n2, 1e-12)


def _top4_indices(s, iota):
    idxs = []
    cur = s
    for _ in range(_K):
        idx = jnp.argmax(cur, axis=1).astype(jnp.int32)[:, None]
        idxs.append(idx)
        cur = jnp.where(iota == idx, -jnp.inf, cur)
    return idxs


def _main_body(emb_ref, wq_ref, k1_ref, k2_ref, s1_ref, s2_ref, q1_ref, grid_ref):
    q = lax.dot_general(emb_ref[...], wq_ref[...], (((1,), (1,)), ((), ())),
                        precision=_PREC, preferred_element_type=jnp.float32)
    q1 = q[:, :_H]
    q2 = q[:, _H:]
    q1 = q1 / jnp.maximum(jnp.sqrt(jnp.sum(q1 * q1, axis=1, keepdims=True)), 1e-12)
    q2 = q2 / jnp.maximum(jnp.sqrt(jnp.sum(q2 * q2, axis=1, keepdims=True)), 1e-12)
    s1 = lax.dot_general(q1, k1_ref[...], (((1,), (1,)), ((), ())),
                         precision=_PREC, preferred_element_type=jnp.float32) / _TEMP
    s2 = lax.dot_general(q2, k2_ref[...], (((1,), (1,)), ((), ())),
                         precision=_PREC, preferred_element_type=jnp.float32) / _TEMP
    s1_ref[...] = s1
    s2_ref[...] = s2
    q1_ref[...] = q1
    iota = lax.broadcasted_iota(jnp.int32, (_BB, _NK), 1)
    rows = _top4_indices(s1, iota)
    cols = _top4_indices(s2, iota)
    parts = []
    for r in range(_K):
        for c in range(_K):
            parts.append(rows[r] * _NK + cols[c])
    grid_ref[...] = jnp.concatenate(parts, axis=1)


def _tc_stage(embedding, Wq, Wrow, Wcol, row_keys, col_keys):
    k1, k2 = pl.pallas_call(
        _keys_body,
        out_shape=[jax.ShapeDtypeStruct((_NK, _H), jnp.float32),
                   jax.ShapeDtypeStruct((_NK, _H), jnp.float32)],
    )(row_keys, col_keys, Wrow, Wcol)

    grid_n = _B // _BB
    s1, s2, q1, grid_idx = pl.pallas_call(
        _main_body,
        grid=(grid_n,),
        in_specs=[
            pl.BlockSpec((_BB, _DIM), lambda i: (i, 0)),
            pl.BlockSpec((_QD, _DIM), lambda i: (0, 0)),
            pl.BlockSpec((_NK, _H), lambda i: (0, 0)),
            pl.BlockSpec((_NK, _H), lambda i: (0, 0)),
        ],
        out_specs=[
            pl.BlockSpec((_BB, _NK), lambda i: (i, 0)),
            pl.BlockSpec((_BB, _NK), lambda i: (i, 0)),
            pl.BlockSpec((_BB, _H), lambda i: (i, 0)),
            pl.BlockSpec((_BB, _NCAND), lambda i: (i, 0)),
        ],
        out_shape=[
            jax.ShapeDtypeStruct((_B, _NK), jnp.float32),
            jax.ShapeDtypeStruct((_B, _NK), jnp.float32),
            jax.ShapeDtypeStruct((_B, _H), jnp.float32),
            jax.ShapeDtypeStruct((_B, _NCAND), jnp.int32),
        ],
        compiler_params=pltpu.CompilerParams(
            dimension_semantics=("parallel",)),
    )(embedding, Wq, k1, k2)
    return s1, s2, q1, grid_idx


def _make_sc_gather():
    info = plsc.get_sparse_core_info()
    nw = info.num_cores * info.num_subcores
    total = _B * _NCAND
    per_w = total // nw
    nch = per_w // 128
    mesh = plsc.VectorSubcoreMesh(core_axis_name="c", subcore_axis_name="s")

    @functools.partial(
        pl.kernel, mesh=mesh,
        out_type=jax.ShapeDtypeStruct((total // 128, 128), jnp.int32),
        scratch_types=[
            pltpu.VMEM((nch, 128), jnp.int32),
            pltpu.VMEM((nch, 128), jnp.int32),
            pltpu.VMEM((nch, 128), jnp.int32),
            pltpu.VMEM((nch, 128), jnp.int32),
            pltpu.SemaphoreType.DMA,
        ],
    )
    def gather_kernel(grid_hbm, offs_hbm, cnts_hbm, inv_hbm, out_hbm,
                      idx_v, off_v, cnt_v, val_v, sem):
        wid = lax.axis_index("s") * info.num_cores + lax.axis_index("c")
        base = wid * nch
        pltpu.sync_copy(grid_hbm.at[pl.ds(base, nch)], idx_v)
        off_pend = []
        cnt_pend = []
        for j in range(nch):
            off_pend.append(pltpu.async_copy(offs_hbm.at[idx_v.at[j]], off_v.at[j], sem))
        for j in range(nch):
            cnt_pend.append(pltpu.async_copy(cnts_hbm.at[idx_v.at[j]], cnt_v.at[j], sem))
        val_pend = []
        for j in range(nch):
            off_pend[j].wait()
            val_pend.append(pltpu.async_copy(inv_hbm.at[off_v.at[j]], val_v.at[j], sem))
        for c in cnt_pend:
            c.wait()
        for c in val_pend:
            c.wait()
        for j in range(nch):
            for t in range(128 // 16):
                sl = pl.ds(t * 16, 16)
                val_v[j, sl] = jnp.where(cnt_v[j, sl] > 0, val_v[j, sl], 0)
        pltpu.sync_copy(val_v, out_hbm.at[pl.ds(base, nch)])

    return gather_kernel


_sc_gather = None


def kernel(embedding, Wq, Wrow, Wcol, row_keys, col_keys,
           cluster_offsets, cluster_counts, inverted_index):
    global _sc_gather
    if _sc_gather is None:
        _sc_gather = _make_sc_gather()
    s1, s2, q1, grid_idx = _tc_stage(embedding, Wq, Wrow, Wcol, row_keys, col_keys)
    grid2 = grid_idx.reshape(_B * _NCAND // 128, 128)
    cand2 = _sc_gather(grid2, cluster_offsets, cluster_counts, inverted_index)
    candidates = cand2.reshape(_B, _NCAND)
    return candidates, s1, s2, q1


# speedup vs baseline: 1.0275x; 1.0275x over previous
import functools

import jax
import jax.numpy as jnp
from jax import lax
from jax.experimental import pallas as pl
from jax.experimental.pallas import tpu as pltpu
from jax.experimental.pallas import tpu_sc as plsc

_B = 4096
_DIM = 4096
_QD = 1024
_H = 512
_NK = 1024
_TEMP = 0.07
_K = 4
_NCAND = 16
_BB = 256

_PREC = lax.Precision.DEFAULT


def _norm_rows(x):
    return x / jnp.maximum(jnp.sqrt(jnp.sum(x * x, axis=1, keepdims=True)), 1e-12)


def _top4_indices(s, iota):
    idxs = []
    cur = s
    for _ in range(_K):
        idx = jnp.argmax(cur, axis=1).astype(jnp.int32)[:, None]
        idxs.append(idx)
        cur = jnp.where(iota == idx, -jnp.inf, cur)
    return idxs


def _main_body(rk_ref, ck_ref, wr_ref, wc_ref, emb_ref, wq_ref,
               s1_ref, s2_ref, q1_ref, grid_ref, k1_s, k2_s):
    @pl.when(pl.program_id(0) == 0)
    def _compute_keys():
        pk1 = lax.dot_general(rk_ref[...], wr_ref[...], (((1,), (1,)), ((), ())),
                              precision=_PREC, preferred_element_type=jnp.float32)
        k1_s[...] = _norm_rows(pk1)
        pk2 = lax.dot_general(ck_ref[...], wc_ref[...], (((1,), (1,)), ((), ())),
                              precision=_PREC, preferred_element_type=jnp.float32)
        k2_s[...] = _norm_rows(pk2)

    q = lax.dot_general(emb_ref[...], wq_ref[...], (((1,), (1,)), ((), ())),
                        precision=_PREC, preferred_element_type=jnp.float32)
    q1 = _norm_rows(q[:, :_H])
    q2 = _norm_rows(q[:, _H:])
    s1 = lax.dot_general(q1, k1_s[...], (((1,), (1,)), ((), ())),
                         precision=_PREC, preferred_element_type=jnp.float32) / _TEMP
    s2 = lax.dot_general(q2, k2_s[...], (((1,), (1,)), ((), ())),
                         precision=_PREC, preferred_element_type=jnp.float32) / _TEMP
    s1_ref[...] = s1
    s2_ref[...] = s2
    q1_ref[...] = q1
    iota = lax.broadcasted_iota(jnp.int32, (_BB, _NK), 1)
    rows = _top4_indices(s1, iota)
    cols = _top4_indices(s2, iota)
    parts = []
    for r in range(_K):
        for c in range(_K):
            parts.append(rows[r] * _NK + cols[c])
    grid_ref[...] = jnp.concatenate(parts, axis=1)


def _tc_stage(embedding, Wq, Wrow, Wcol, row_keys, col_keys):
    grid_n = _B // _BB
    s1, s2, q1, grid_idx = pl.pallas_call(
        _main_body,
        grid=(grid_n,),
        in_specs=[
            pl.BlockSpec((_NK, _H), lambda i: (0, 0)),
            pl.BlockSpec((_NK, _H), lambda i: (0, 0)),
            pl.BlockSpec((_H, _H), lambda i: (0, 0)),
            pl.BlockSpec((_H, _H), lambda i: (0, 0)),
            pl.BlockSpec((_BB, _DIM), lambda i: (i, 0)),
            pl.BlockSpec((_QD, _DIM), lambda i: (0, 0)),
        ],
        out_specs=[
            pl.BlockSpec((_BB, _NK), lambda i: (i, 0)),
            pl.BlockSpec((_BB, _NK), lambda i: (i, 0)),
            pl.BlockSpec((_BB, _H), lambda i: (i, 0)),
            pl.BlockSpec((_BB, _NCAND), lambda i: (i, 0)),
        ],
        out_shape=[
            jax.ShapeDtypeStruct((_B, _NK), jnp.float32),
            jax.ShapeDtypeStruct((_B, _NK), jnp.float32),
            jax.ShapeDtypeStruct((_B, _H), jnp.float32),
            jax.ShapeDtypeStruct((_B, _NCAND), jnp.int32),
        ],
        scratch_shapes=[
            pltpu.VMEM((_NK, _H), jnp.float32),
            pltpu.VMEM((_NK, _H), jnp.float32),
        ],
        compiler_params=pltpu.CompilerParams(
            dimension_semantics=("arbitrary",)),
    )(row_keys, col_keys, Wrow, Wcol, embedding, Wq)
    return s1, s2, q1, grid_idx


def _make_sc_gather():
    info = plsc.get_sparse_core_info()
    nw = info.num_cores * info.num_subcores
    total = _B * _NCAND
    per_w = total // nw
    nch = per_w // 128
    mesh = plsc.VectorSubcoreMesh(core_axis_name="c", subcore_axis_name="s")

    @functools.partial(
        pl.kernel, mesh=mesh,
        out_type=jax.ShapeDtypeStruct((total // 128, 128), jnp.int32),
        scratch_types=[
            pltpu.VMEM((nch, 128), jnp.int32),
            pltpu.VMEM((nch, 128), jnp.int32),
            pltpu.VMEM((nch, 128), jnp.int32),
            pltpu.VMEM((nch, 128), jnp.int32),
            pltpu.SemaphoreType.DMA,
        ],
    )
    def gather_kernel(grid_hbm, offs_hbm, cnts_hbm, inv_hbm, out_hbm,
                      idx_v, off_v, cnt_v, val_v, sem):
        wid = lax.axis_index("s") * info.num_cores + lax.axis_index("c")
        base = wid * nch
        pltpu.sync_copy(grid_hbm.at[pl.ds(base, nch)], idx_v)
        off_pend = []
        cnt_pend = []
        for j in range(nch):
            off_pend.append(pltpu.async_copy(offs_hbm.at[idx_v.at[j]], off_v.at[j], sem))
        for j in range(nch):
            cnt_pend.append(pltpu.async_copy(cnts_hbm.at[idx_v.at[j]], cnt_v.at[j], sem))
        val_pend = []
        for j in range(nch):
            off_pend[j].wait()
            val_pend.append(pltpu.async_copy(inv_hbm.at[off_v.at[j]], val_v.at[j], sem))
        for c in cnt_pend:
            c.wait()
        for c in val_pend:
            c.wait()
        for j in range(nch):
            for t in range(128 // 16):
                sl = pl.ds(t * 16, 16)
                val_v[j, sl] = jnp.where(cnt_v[j, sl] > 0, val_v[j, sl], 0)
        pltpu.sync_copy(val_v, out_hbm.at[pl.ds(base, nch)])

    return gather_kernel


_sc_gather = None


def kernel(embedding, Wq, Wrow, Wcol, row_keys, col_keys,
           cluster_offsets, cluster_counts, inverted_index):
    global _sc_gather
    if _sc_gather is None:
        _sc_gather = _make_sc_gather()
    s1, s2, q1, grid_idx = _tc_stage(embedding, Wq, Wrow, Wcol, row_keys, col_keys)
    grid2 = grid_idx.reshape(_B * _NCAND // 128, 128)
    cand2 = _sc_gather(grid2, cluster_offsets, cluster_counts, inverted_index)
    candidates = cand2.reshape(_B, _NCAND)
    return candidates, s1, s2, q1


# speedup vs baseline: 1.0353x; 1.0076x over previous
import functools

import jax
import jax.numpy as jnp
from jax import lax
from jax.experimental import pallas as pl
from jax.experimental.pallas import tpu as pltpu
from jax.experimental.pallas import tpu_sc as plsc

_B = 4096
_DIM = 4096
_QD = 1024
_H = 512
_NK = 1024
_TEMP = 0.07
_K = 4
_NCAND = 16
_BB = 512

_PREC = lax.Precision.DEFAULT


def _norm_rows(x):
    return x / jnp.maximum(jnp.sqrt(jnp.sum(x * x, axis=1, keepdims=True)), 1e-12)


def _top4_indices(s, iota):
    idxs = []
    cur = s
    for _ in range(_K):
        idx = jnp.argmax(cur, axis=1).astype(jnp.int32)[:, None]
        idxs.append(idx)
        cur = jnp.where(iota == idx, -jnp.inf, cur)
    return idxs


def _main_body(rk_ref, ck_ref, wr_ref, wc_ref, emb_ref, wq_ref,
               s1_ref, s2_ref, q1_ref, grid_ref, k1_s, k2_s):
    @pl.when(pl.program_id(0) == 0)
    def _compute_keys():
        pk1 = lax.dot_general(rk_ref[...], wr_ref[...], (((1,), (1,)), ((), ())),
                              precision=_PREC, preferred_element_type=jnp.float32)
        k1_s[...] = _norm_rows(pk1)
        pk2 = lax.dot_general(ck_ref[...], wc_ref[...], (((1,), (1,)), ((), ())),
                              precision=_PREC, preferred_element_type=jnp.float32)
        k2_s[...] = _norm_rows(pk2)

    q = lax.dot_general(emb_ref[...], wq_ref[...], (((1,), (1,)), ((), ())),
                        precision=_PREC, preferred_element_type=jnp.float32)
    q1 = _norm_rows(q[:, :_H])
    q2 = _norm_rows(q[:, _H:])
    s1 = lax.dot_general(q1, k1_s[...], (((1,), (1,)), ((), ())),
                         precision=_PREC, preferred_element_type=jnp.float32) / _TEMP
    s2 = lax.dot_general(q2, k2_s[...], (((1,), (1,)), ((), ())),
                         precision=_PREC, preferred_element_type=jnp.float32) / _TEMP
    s1_ref[...] = s1
    s2_ref[...] = s2
    q1_ref[...] = q1
    iota = lax.broadcasted_iota(jnp.int32, (_BB, _NK), 1)
    rows = _top4_indices(s1, iota)
    cols = _top4_indices(s2, iota)
    parts = []
    for r in range(_K):
        for c in range(_K):
            parts.append(rows[r] * _NK + cols[c])
    grid_ref[...] = jnp.concatenate(parts, axis=1)


def _tc_stage(embedding, Wq, Wrow, Wcol, row_keys, col_keys):
    grid_n = _B // _BB
    s1, s2, q1, grid_idx = pl.pallas_call(
        _main_body,
        grid=(grid_n,),
        in_specs=[
            pl.BlockSpec((_NK, _H), lambda i: (0, 0)),
            pl.BlockSpec((_NK, _H), lambda i: (0, 0)),
            pl.BlockSpec((_H, _H), lambda i: (0, 0)),
            pl.BlockSpec((_H, _H), lambda i: (0, 0)),
            pl.BlockSpec((_BB, _DIM), lambda i: (i, 0)),
            pl.BlockSpec((_QD, _DIM), lambda i: (0, 0)),
        ],
        out_specs=[
            pl.BlockSpec((_BB, _NK), lambda i: (i, 0)),
            pl.BlockSpec((_BB, _NK), lambda i: (i, 0)),
            pl.BlockSpec((_BB, _H), lambda i: (i, 0)),
            pl.BlockSpec((_BB, _NCAND), lambda i: (i, 0)),
        ],
        out_shape=[
            jax.ShapeDtypeStruct((_B, _NK), jnp.float32),
            jax.ShapeDtypeStruct((_B, _NK), jnp.float32),
            jax.ShapeDtypeStruct((_B, _H), jnp.float32),
            jax.ShapeDtypeStruct((_B, _NCAND), jnp.int32),
        ],
        scratch_shapes=[
            pltpu.VMEM((_NK, _H), jnp.float32),
            pltpu.VMEM((_NK, _H), jnp.float32),
        ],
        compiler_params=pltpu.CompilerParams(
            dimension_semantics=("arbitrary",),
            vmem_limit_bytes=100 * 1024 * 1024),
    )(row_keys, col_keys, Wrow, Wcol, embedding, Wq)
    return s1, s2, q1, grid_idx


def _make_sc_gather():
    info = plsc.get_sparse_core_info()
    nw = info.num_cores * info.num_subcores
    total = _B * _NCAND
    per_w = total // nw
    nch = per_w // 128
    mesh = plsc.VectorSubcoreMesh(core_axis_name="c", subcore_axis_name="s")

    @functools.partial(
        pl.kernel, mesh=mesh,
        out_type=jax.ShapeDtypeStruct((total // 128, 128), jnp.int32),
        scratch_types=[
            pltpu.VMEM((nch, 128), jnp.int32),
            pltpu.VMEM((nch, 128), jnp.int32),
            pltpu.VMEM((nch, 128), jnp.int32),
            pltpu.VMEM((nch, 128), jnp.int32),
            pltpu.SemaphoreType.DMA,
        ],
    )
    def gather_kernel(grid_hbm, offs_hbm, cnts_hbm, inv_hbm, out_hbm,
                      idx_v, off_v, cnt_v, val_v, sem):
        wid = lax.axis_index("s") * info.num_cores + lax.axis_index("c")
        base = wid * nch
        pltpu.sync_copy(grid_hbm.at[pl.ds(base, nch)], idx_v)
        off_pend = []
        cnt_pend = []
        for j in range(nch):
            off_pend.append(pltpu.async_copy(offs_hbm.at[idx_v.at[j]], off_v.at[j], sem))
        for j in range(nch):
            cnt_pend.append(pltpu.async_copy(cnts_hbm.at[idx_v.at[j]], cnt_v.at[j], sem))
        val_pend = []
        for j in range(nch):
            off_pend[j].wait()
            val_pend.append(pltpu.async_copy(inv_hbm.at[off_v.at[j]], val_v.at[j], sem))
        for c in cnt_pend:
            c.wait()
        for c in val_pend:
            c.wait()
        for j in range(nch):
            for t in range(128 // 16):
                sl = pl.ds(t * 16, 16)
                val_v[j, sl] = jnp.where(cnt_v[j, sl] > 0, val_v[j, sl], 0)
        pltpu.sync_copy(val_v, out_hbm.at[pl.ds(base, nch)])

    return gather_kernel


_sc_gather = None


def kernel(embedding, Wq, Wrow, Wcol, row_keys, col_keys,
           cluster_offsets, cluster_counts, inverted_index):
    global _sc_gather
    if _sc_gather is None:
        _sc_gather = _make_sc_gather()
    s1, s2, q1, grid_idx = _tc_stage(embedding, Wq, Wrow, Wcol, row_keys, col_keys)
    grid2 = grid_idx.reshape(_B * _NCAND // 128, 128)
    cand2 = _sc_gather(grid2, cluster_offsets, cluster_counts, inverted_index)
    candidates = cand2.reshape(_B, _NCAND)
    return candidates, s1, s2, q1


# speedup vs baseline: 1.0356x; 1.0003x over previous
import functools

import jax
import jax.numpy as jnp
from jax import lax
from jax.experimental import pallas as pl
from jax.experimental.pallas import tpu as pltpu
from jax.experimental.pallas import tpu_sc as plsc

_B = 4096
_DIM = 4096
_QD = 1024
_H = 512
_NK = 1024
_TEMP = 0.07
_K = 4
_NCAND = 16
_BB = 512

_PREC = lax.Precision.DEFAULT


def _norm_rows(x):
    return x / jnp.maximum(jnp.sqrt(jnp.sum(x * x, axis=1, keepdims=True)), 1e-12)


def _top4_indices(s, iota):
    idxs = []
    cur = s
    for _ in range(_K):
        idx = jnp.argmax(cur, axis=1).astype(jnp.int32)[:, None]
        idxs.append(idx)
        cur = jnp.where(iota == idx, -jnp.inf, cur)
    return idxs


def _main_body(rk_ref, ck_ref, wr_ref, wc_ref, emb_ref, wq_ref,
               s1_ref, s2_ref, q1_ref, grid_ref, k1_s, k2_s):
    @pl.when(pl.program_id(0) == 0)
    def _compute_keys():
        pk1 = lax.dot_general(rk_ref[...], wr_ref[...], (((1,), (1,)), ((), ())),
                              precision=_PREC, preferred_element_type=jnp.float32)
        k1_s[...] = _norm_rows(pk1)
        pk2 = lax.dot_general(ck_ref[...], wc_ref[...], (((1,), (1,)), ((), ())),
                              precision=_PREC, preferred_element_type=jnp.float32)
        k2_s[...] = _norm_rows(pk2)

    q = lax.dot_general(emb_ref[...], wq_ref[...], (((1,), (1,)), ((), ())),
                        precision=_PREC, preferred_element_type=jnp.float32)
    q1 = _norm_rows(q[:, :_H])
    q2 = _norm_rows(q[:, _H:])
    s1 = lax.dot_general(q1, k1_s[...], (((1,), (1,)), ((), ())),
                         precision=_PREC, preferred_element_type=jnp.float32) / _TEMP
    s2 = lax.dot_general(q2, k2_s[...], (((1,), (1,)), ((), ())),
                         precision=_PREC, preferred_element_type=jnp.float32) / _TEMP
    s1_ref[...] = s1
    s2_ref[...] = s2
    q1_ref[...] = q1
    iota = lax.broadcasted_iota(jnp.int32, (_BB, _NK), 1)
    rows = _top4_indices(s1, iota)
    cols = _top4_indices(s2, iota)
    parts = []
    for r in range(_K):
        for c in range(_K):
            parts.append(rows[r] * _NK + cols[c])
    grid_ref[...] = jnp.concatenate(parts, axis=1)


def _tc_stage(embedding, Wq, Wrow, Wcol, row_keys, col_keys):
    grid_n = _B // _BB
    s1, s2, q1, grid_idx = pl.pallas_call(
        _main_body,
        grid=(grid_n,),
        in_specs=[
            pl.BlockSpec((_NK, _H), lambda i: (0, 0)),
            pl.BlockSpec((_NK, _H), lambda i: (0, 0)),
            pl.BlockSpec((_H, _H), lambda i: (0, 0)),
            pl.BlockSpec((_H, _H), lambda i: (0, 0)),
            pl.BlockSpec((_BB, _DIM), lambda i: (i, 0)),
            pl.BlockSpec((_QD, _DIM), lambda i: (0, 0)),
        ],
        out_specs=[
            pl.BlockSpec((_BB, _NK), lambda i: (i, 0)),
            pl.BlockSpec((_BB, _NK), lambda i: (i, 0)),
            pl.BlockSpec((_BB, _H), lambda i: (i, 0)),
            pl.BlockSpec((_BB, _NCAND), lambda i: (i, 0)),
        ],
        out_shape=[
            jax.ShapeDtypeStruct((_B, _NK), jnp.float32),
            jax.ShapeDtypeStruct((_B, _NK), jnp.float32),
            jax.ShapeDtypeStruct((_B, _H), jnp.float32),
            jax.ShapeDtypeStruct((_B, _NCAND), jnp.int32),
        ],
        scratch_shapes=[
            pltpu.VMEM((_NK, _H), jnp.float32),
            pltpu.VMEM((_NK, _H), jnp.float32),
        ],
        compiler_params=pltpu.CompilerParams(
            dimension_semantics=("arbitrary",),
            vmem_limit_bytes=100 * 1024 * 1024),
    )(row_keys, col_keys, Wrow, Wcol, embedding, Wq)
    return s1, s2, q1, grid_idx


def _make_sc_gather():
    info = plsc.get_sparse_core_info()
    nw = info.num_cores * info.num_subcores
    total = _B * _NCAND
    per_w = total // nw
    nch = per_w // 128
    mesh = plsc.VectorSubcoreMesh(core_axis_name="c", subcore_axis_name="s")

    @functools.partial(
        pl.kernel, mesh=mesh,
        out_type=jax.ShapeDtypeStruct((total // 128, 128), jnp.int32),
        scratch_types=[
            pltpu.VMEM((nch, 128), jnp.int32),
            pltpu.VMEM((nch, 128), jnp.int32),
            pltpu.VMEM((nch, 128), jnp.int32),
            pltpu.VMEM((nch, 128), jnp.int32),
            pltpu.SemaphoreType.DMA,
        ],
    )
    def gather_kernel(grid_hbm, offs_hbm, cnts_hbm, inv_hbm, out_hbm,
                      idx_v, off_v, cnt_v, val_v, sem):
        wid = lax.axis_index("s") * info.num_cores + lax.axis_index("c")
        base = wid * nch
        pltpu.sync_copy(grid_hbm.at[pl.ds(base, nch)], idx_v)
        off_pend = []
        cnt_pend = []
        for j in range(nch):
            off_pend.append(pltpu.async_copy(offs_hbm.at[idx_v.at[j]], off_v.at[j], sem))
        for j in range(nch):
            cnt_pend.append(pltpu.async_copy(cnts_hbm.at[idx_v.at[j]], cnt_v.at[j], sem))
        val_pend = []
        for j in range(nch):
            off_pend[j].wait()
            val_pend.append(pltpu.async_copy(inv_hbm.at[off_v.at[j]], val_v.at[j], sem))
        for c in cnt_pend:
            c.wait()
        out_pend = []
        for j in range(nch):
            val_pend[j].wait()
            for t in range(128 // 16):
                sl = pl.ds(t * 16, 16)
                val_v[j, sl] = jnp.where(cnt_v[j, sl] > 0, val_v[j, sl], 0)
            out_pend.append(pltpu.async_copy(val_v.at[j], out_hbm.at[base + j], sem))
        for c in out_pend:
            c.wait()

    return gather_kernel


_sc_gather = None


def kernel(embedding, Wq, Wrow, Wcol, row_keys, col_keys,
           cluster_offsets, cluster_counts, inverted_index):
    global _sc_gather
    if _sc_gather is None:
        _sc_gather = _make_sc_gather()
    s1, s2, q1, grid_idx = _tc_stage(embedding, Wq, Wrow, Wcol, row_keys, col_keys)
    grid2 = grid_idx.reshape(_B * _NCAND // 128, 128)
    cand2 = _sc_gather(grid2, cluster_offsets, cluster_counts, inverted_index)
    candidates = cand2.reshape(_B, _NCAND)
    return candidates, s1, s2, q1
